# baseline (speedup 1.0000x reference)
import functools
import math

import jax
import jax.numpy as jnp
from jax.experimental import pallas as pl
from jax.experimental.pallas import tpu as pltpu

_NIH = 2
_DIDX = 32
_TOPK = 256
_RB = 256


def _indexer_body(x_ref, w_ref, b_ref, g_ref, gt_ref, gam_ref, bet_ref, o_ref):
    xw = jnp.dot(x_ref[...].astype(jnp.bfloat16),
                 w_ref[...].astype(jnp.bfloat16),
                 preferred_element_type=jnp.float32)
    xw = xw + b_ref[...]
    inv_d = 1.0 / _DIDX
    mean = jnp.dot(xw, g_ref[...], preferred_element_type=jnp.float32, precision=jax.lax.Precision.HIGHEST) * inv_d
    msq = jnp.dot(xw * xw, g_ref[...], preferred_element_type=jnp.float32, precision=jax.lax.Precision.HIGHEST) * inv_d
    var = msq - mean * mean
    meanb = jnp.dot(mean, gt_ref[...], preferred_element_type=jnp.float32, precision=jax.lax.Precision.HIGHEST)
    varb = jnp.dot(var, gt_ref[...], preferred_element_type=jnp.float32, precision=jax.lax.Precision.HIGHEST)
    inv = jax.lax.rsqrt(varb + 1e-5)
    o_ref[...] = (xw - meanb) * inv * gam_ref[...] + bet_ref[...]


def _round_bf16(x):
    i = jax.lax.bitcast_convert_type(x, jnp.int32)
    i = (i + 0x7FFF + ((i >> 16) & 1)) & jnp.int32(-65536)
    return jax.lax.bitcast_convert_type(i, jnp.float32)


def _attn_body(qkb_ref, qka_ref, qb_ref, k_ref, ve_ref, w_ref, o_ref,
               bias_ref, *, S, H, DK, kt):

    dn = (((1,), (1,)), ((), ()))
    kb = qkb_ref[...]
    qa = qka_ref[...]
    k0 = kb[:, 2 * _DIDX:3 * _DIDX]
    k1 = kb[:, 3 * _DIDX:4 * _DIDX]
    q0 = qa[:, 0:_DIDX]
    q1 = qa[:, _DIDX:2 * _DIDX]
    w = w_ref[...]
    dn = (((1,), (1,)), ((), ()))
    a0 = jax.lax.dot_general(k0.astype(jnp.bfloat16), q0.astype(jnp.bfloat16),
                             dn, preferred_element_type=jnp.float32)
    a1 = jax.lax.dot_general(k1.astype(jnp.bfloat16), q1.astype(jnp.bfloat16),
                             dn, preferred_element_type=jnp.float32)
    r0 = _round_bf16(jnp.maximum(a0, 0.0))
    r1 = _round_bf16(jnp.maximum(a1, 0.0))
    wb = _round_bf16(w)
    A = r0 * wb[:, 0:1] + r1 * wb[:, 1:2]

    Ai = jax.lax.bitcast_convert_type(A, jnp.int32)
    row_max = jnp.max(Ai, axis=1, keepdims=True)
    kt_f = jnp.float32(kt)

    def bs_step(lo, hi):
        mid = lo + ((hi - lo) >> 1)
        cnt = jnp.sum(jnp.where(Ai >= mid, 1.0, 0.0), axis=1, keepdims=True)
        ge = cnt >= kt_f
        return jnp.where(ge, mid, lo), jnp.where(ge, hi, mid)

    zero = jnp.zeros_like(row_max)
    scale = 1.0 / math.sqrt(DK)
    kb16 = k_ref[...].astype(jnp.bfloat16)
    ve16 = ve_ref[...].astype(jnp.bfloat16)

    def head_body(h, carry):
        lo, hi = carry
        qh = (qb_ref[h] * scale).astype(jnp.bfloat16)
        s = jax.lax.dot_general(qh, kb16, dn,
                                preferred_element_type=jnp.float32)
        p = jnp.exp(s + bias_ref[...])
        oe = jnp.dot(p.astype(jnp.bfloat16), ve16,
                     preferred_element_type=jnp.float32)
        o_ref[h] = oe[:, 0:DK] / oe[:, DK:DK + 1]
        lo, hi = bs_step(lo, hi)
        return bs_step(lo, hi)

    carry = (zero, row_max + 1)
    if 2 * H < 31:
        carry = jax.lax.fori_loop(
            0, 31 - 2 * H, lambda _, c: bs_step(*c), carry)
    v, _ = jax.lax.fori_loop(0, H, head_body, carry)

    n_gt = jnp.sum(jnp.where(Ai > v, 1.0, 0.0), axis=1, keepdims=True)
    m = kt_f - n_gt
    tie = Ai == v
    iota = jax.lax.broadcasted_iota(jnp.int32, A.shape, 1)

    def ix_body(_, carry):
        lo, hi = carry
        mid = lo + ((hi - lo) >> 1)
        cnt = jnp.sum(jnp.where(tie & (iota < mid), 1.0, 0.0),
                      axis=1, keepdims=True)
        ge = cnt >= m
        return jnp.where(ge, lo, mid), jnp.where(ge, mid, hi)

    _, cut = jax.lax.fori_loop(0, 11, ix_body,
                               (zero, jnp.full_like(row_max, S)))
    mask = (Ai > v) | (tie & (iota < cut))
    bias_ref[...] = jnp.where(mask, 0.0, -1e30)


def kernel(x, Q, K, V, Wq_idx, bq_idx, Wk_idx, bk_idx, ln_gamma, ln_beta,
           indexer_weights):
    B, S, DM = x.shape
    H, DK = Q.shape[1], Q.shape[3]
    C = 2 * _NIH * _DIDX
    kt = min(_TOPK, S)

    x2 = x.reshape(S, DM)
    W = jnp.concatenate([Wq_idx, Wk_idx], axis=1)
    b = jnp.concatenate([bq_idx, bk_idx]).reshape(1, C)
    gam = jnp.tile(ln_gamma, 2 * _NIH).reshape(1, C)
    bet = jnp.tile(ln_beta, 2 * _NIH).reshape(1, C)
    G = jnp.repeat(jnp.eye(2 * _NIH, dtype=jnp.float32), _DIDX, axis=0)
    GT = G.T

    nb = S // _RB
    qk = pl.pallas_call(
        _indexer_body,
        grid=(nb,),
        in_specs=[
            pl.BlockSpec((_RB, DM), lambda i: (i, 0)),
            pl.BlockSpec((DM, C), lambda i: (0, 0)),
            pl.BlockSpec((1, C), lambda i: (0, 0)),
            pl.BlockSpec((C, 2 * _NIH), lambda i: (0, 0)),
            pl.BlockSpec((2 * _NIH, C), lambda i: (0, 0)),
            pl.BlockSpec((1, C), lambda i: (0, 0)),
            pl.BlockSpec((1, C), lambda i: (0, 0)),
        ],
        out_specs=pl.BlockSpec((_RB, C), lambda i: (i, 0)),
        out_shape=jax.ShapeDtypeStruct((S, C), jnp.float32),
    )(x2, W, b, G, GT, gam, bet)

    Qr = Q.reshape(H, S, DK)
    K2 = K.reshape(S, DK)
    Ve = jnp.concatenate(
        [V.reshape(S, DK), jnp.ones((S, 1), jnp.float32)], axis=1)
    wv = indexer_weights.reshape(1, _NIH)

    out = pl.pallas_call(
        functools.partial(_attn_body, S=S, H=H, DK=DK, kt=kt),
        grid=(nb + 1,),
        in_specs=[
            pl.BlockSpec((_RB, C), lambda i: (jnp.minimum(i, nb - 1), 0)),
            pl.BlockSpec((S, C), lambda i: (0, 0)),
            pl.BlockSpec((H, _RB, DK), lambda i: (0, jnp.maximum(i - 1, 0), 0)),
            pl.BlockSpec((S, DK), lambda i: (0, 0)),
            pl.BlockSpec((S, DK + 1), lambda i: (0, 0)),
            pl.BlockSpec((1, _NIH), lambda i: (0, 0)),
        ],
        out_specs=pl.BlockSpec((H, _RB, DK),
                               lambda i: (0, jnp.maximum(i - 1, 0), 0)),
        out_shape=jax.ShapeDtypeStruct((H, S, DK), jnp.float32),
        scratch_shapes=[pltpu.VMEM((_RB, S), jnp.float32)],
    )(qk, qk, Qr, K2, Ve, wv)

    return out.transpose(1, 0, 2).reshape(B, S, H * DK), jnp.float32(0.0)


# speedup vs baseline: 1.2293x; 1.2293x over previous
import functools
import math

import jax
import jax.numpy as jnp
from jax.experimental import pallas as pl

_NIH = 2
_DIDX = 32
_TOPK = 256
_RB = 256


def _indexer_body(x_ref, w_ref, b_ref, g_ref, gt_ref, gam_ref, bet_ref, o_ref):
    xw = jnp.dot(x_ref[...].astype(jnp.bfloat16),
                 w_ref[...].astype(jnp.bfloat16),
                 preferred_element_type=jnp.float32)
    xw = xw + b_ref[...]
    inv_d = 1.0 / _DIDX
    hi = jax.lax.Precision.HIGHEST
    mean = jnp.dot(xw, g_ref[...], preferred_element_type=jnp.float32,
                   precision=hi) * inv_d
    msq = jnp.dot(xw * xw, g_ref[...], preferred_element_type=jnp.float32,
                  precision=hi) * inv_d
    var = msq - mean * mean
    meanb = jnp.dot(mean, gt_ref[...], preferred_element_type=jnp.float32,
                    precision=hi)
    varb = jnp.dot(var, gt_ref[...], preferred_element_type=jnp.float32,
                   precision=hi)
    inv = jax.lax.rsqrt(varb + 1e-5)
    o_ref[...] = (xw - meanb) * inv * gam_ref[...] + bet_ref[...]


def _round_bf16(x):
    i = jax.lax.bitcast_convert_type(x, jnp.int32)
    i = (i + 0x7FFF + ((i >> 16) & 1)) & jnp.int32(-65536)
    return jax.lax.bitcast_convert_type(i, jnp.float32)


def _attn_body(qkb_ref, qka_ref, qb_ref, k_ref, ve_ref, w_ref, o_ref, *,
               S, H, DK, kt):
    kb = qkb_ref[...]
    qa = qka_ref[...]
    k0 = kb[:, 2 * _DIDX:3 * _DIDX]
    k1 = kb[:, 3 * _DIDX:4 * _DIDX]
    q0 = qa[:, 0:_DIDX]
    q1 = qa[:, _DIDX:2 * _DIDX]
    w = w_ref[...]
    dn = (((1,), (1,)), ((), ()))
    a0 = jax.lax.dot_general(k0.astype(jnp.bfloat16), q0.astype(jnp.bfloat16),
                             dn, preferred_element_type=jnp.float32)
    a1 = jax.lax.dot_general(k1.astype(jnp.bfloat16), q1.astype(jnp.bfloat16),
                             dn, preferred_element_type=jnp.float32)
    r0 = _round_bf16(jnp.maximum(a0, 0.0))
    r1 = _round_bf16(jnp.maximum(a1, 0.0))
    wb = _round_bf16(w)
    A = r0 * wb[:, 0:1] + r1 * wb[:, 1:2]

    Ai = jax.lax.bitcast_convert_type(A, jnp.int32)
    row_max = jnp.max(Ai, axis=1, keepdims=True)
    kt_f = jnp.float32(kt)

    def bs_body(_, carry):
        lo, hi = carry
        mid = lo + ((hi - lo) >> 1)
        cnt = jnp.sum(jnp.where(Ai >= mid, 1.0, 0.0), axis=1, keepdims=True)
        ge = cnt >= kt_f
        return jnp.where(ge, mid, lo), jnp.where(ge, hi, mid)

    zero = jnp.zeros_like(row_max)
    v, _ = jax.lax.fori_loop(0, 31, bs_body, (zero, row_max + 1))

    n_gt = jnp.sum(jnp.where(Ai > v, 1.0, 0.0), axis=1, keepdims=True)
    m = kt_f - n_gt
    tie = Ai == v
    iota = jax.lax.broadcasted_iota(jnp.int32, A.shape, 1)

    def ix_body(_, carry):
        lo, hi = carry
        mid = lo + ((hi - lo) >> 1)
        cnt = jnp.sum(jnp.where(tie & (iota < mid), 1.0, 0.0),
                      axis=1, keepdims=True)
        ge = cnt >= m
        return jnp.where(ge, lo, mid), jnp.where(ge, mid, hi)

    _, cut = jax.lax.fori_loop(0, 11, ix_body,
                               (zero, jnp.full_like(row_max, S)))
    mask = (Ai > v) | (tie & (iota < cut))
    bias = jnp.where(mask, 0.0, -1e30)

    scale = 1.0 / math.sqrt(DK)
    kb16 = k_ref[...].astype(jnp.bfloat16)
    ve16 = ve_ref[...].astype(jnp.bfloat16)
    outs = []
    for h in range(H):
        qh = (qb_ref[h] * scale).astype(jnp.bfloat16)
        s = jax.lax.dot_general(qh, kb16, dn,
                                preferred_element_type=jnp.float32)
        p = jnp.exp(s + bias)
        oe = jnp.dot(p.astype(jnp.bfloat16), ve16,
                     preferred_element_type=jnp.float32)
        outs.append(oe[:, 0:DK] / oe[:, DK:DK + 1])
    o_ref[...] = jnp.concatenate(outs, axis=1)


def kernel(x, Q, K, V, Wq_idx, bq_idx, Wk_idx, bk_idx, ln_gamma, ln_beta,
           indexer_weights):
    B, S, DM = x.shape
    H, DK = Q.shape[1], Q.shape[3]
    C = 2 * _NIH * _DIDX
    kt = min(_TOPK, S)

    x2 = x.reshape(S, DM)
    W = jnp.concatenate([Wq_idx, Wk_idx], axis=1)
    b = jnp.concatenate([bq_idx, bk_idx]).reshape(1, C)
    gam = jnp.tile(ln_gamma, 2 * _NIH).reshape(1, C)
    bet = jnp.tile(ln_beta, 2 * _NIH).reshape(1, C)
    G = jnp.repeat(jnp.eye(2 * _NIH, dtype=jnp.float32), _DIDX, axis=0)
    GT = G.T

    nb = S // _RB
    qk = pl.pallas_call(
        _indexer_body,
        grid=(nb,),
        in_specs=[
            pl.BlockSpec((_RB, DM), lambda i: (i, 0)),
            pl.BlockSpec((DM, C), lambda i: (0, 0)),
            pl.BlockSpec((1, C), lambda i: (0, 0)),
            pl.BlockSpec((C, 2 * _NIH), lambda i: (0, 0)),
            pl.BlockSpec((2 * _NIH, C), lambda i: (0, 0)),
            pl.BlockSpec((1, C), lambda i: (0, 0)),
            pl.BlockSpec((1, C), lambda i: (0, 0)),
        ],
        out_specs=pl.BlockSpec((_RB, C), lambda i: (i, 0)),
        out_shape=jax.ShapeDtypeStruct((S, C), jnp.float32),
    )(x2, W, b, G, GT, gam, bet)

    Qr = Q.reshape(H, S, DK)
    K2 = K.reshape(S, DK)
    Ve = jnp.concatenate(
        [V.reshape(S, DK), jnp.ones((S, 1), jnp.float32)], axis=1)
    wv = indexer_weights.reshape(1, _NIH)

    out = pl.pallas_call(
        functools.partial(_attn_body, S=S, H=H, DK=DK, kt=kt),
        grid=(nb,),
        in_specs=[
            pl.BlockSpec((_RB, C), lambda i: (i, 0)),
            pl.BlockSpec((S, C), lambda i: (0, 0)),
            pl.BlockSpec((H, _RB, DK), lambda i: (0, i, 0)),
            pl.BlockSpec((S, DK), lambda i: (0, 0)),
            pl.BlockSpec((S, DK + 1), lambda i: (0, 0)),
            pl.BlockSpec((1, _NIH), lambda i: (0, 0)),
        ],
        out_specs=pl.BlockSpec((_RB, H * DK), lambda i: (i, 0)),
        out_shape=jax.ShapeDtypeStruct((S, H * DK), jnp.float32),
    )(qk, qk, Qr, K2, Ve, wv)

    return out.reshape(B, S, H * DK), jnp.float32(0.0)
